# Initial kernel scaffold; baseline (speedup 1.0000x reference)
import functools

import jax
import jax.numpy as jnp
from jax import lax
from jax.experimental import pallas as pl
from jax.experimental.pallas import tpu as pltpu
from jax.experimental.pallas import tpu_sc as plsc

NC = 2
NS = 16
EPR = 128


def _mesh():
    return plsc.VectorSubcoreMesh(
        core_axis_name="c", subcore_axis_name="s", num_cores=NC, num_subcores=NS
    )


def _sc_degree(dst_r, zeros16, ones16, r_pad, rows_per_sub):
    k_steps = dst_r.shape[2]

    @functools.partial(
        pl.kernel,
        out_type=jax.ShapeDtypeStruct((NC, r_pad, 16), jnp.float32),
        mesh=_mesh(),
        scratch_types=[
            pltpu.VMEM((k_steps, EPR), jnp.int32),
            pltpu.VMEM((EPR, 16), jnp.float32),
            pltpu.VMEM_SHARED((r_pad, 16), jnp.float32),
        ],
    )
    def deg_kernel(dst_hbm, z_hbm, one_hbm, out_hbm, dst_v, one_v, acc_sh):
        cid = lax.axis_index("c")
        sid = lax.axis_index("s")
        row0 = sid * rows_per_sub
        pltpu.sync_copy(z_hbm, acc_sh.at[pl.ds(row0, rows_per_sub)])
        pltpu.sync_copy(one_hbm, one_v)
        pltpu.sync_copy(dst_hbm.at[cid, sid], dst_v)
        plsc.subcore_barrier()

        @pl.loop(0, k_steps)
        def _(j):
            pltpu.sync_copy(one_v, acc_sh.at[dst_v.at[j]], add=True)

        plsc.subcore_barrier()
        pltpu.sync_copy(
            acc_sh.at[pl.ds(row0, rows_per_sub)],
            out_hbm.at[cid, pl.ds(row0, rows_per_sub)],
        )

    return deg_kernel(dst_r, zeros16, ones16)


def _sc_aggregate(g, src_r, dst_r, zeros_d, r_pad, rows_per_sub):
    d = g.shape[1]
    k_steps = src_r.shape[2]

    @functools.partial(
        pl.kernel,
        out_type=jax.ShapeDtypeStruct((NC, r_pad, d), jnp.float32),
        mesh=_mesh(),
        scratch_types=[
            pltpu.VMEM((k_steps, EPR), jnp.int32),
            pltpu.VMEM((k_steps, EPR), jnp.int32),
            pltpu.VMEM((EPR, d), jnp.float32),
            pltpu.VMEM_SHARED((r_pad, d), jnp.float32),
        ],
    )
    def agg_kernel(g_hbm, src_hbm, dst_hbm, z_hbm, out_hbm,
                   src_v, dst_v, gbuf, acc_sh):
        cid = lax.axis_index("c")
        sid = lax.axis_index("s")
        row0 = sid * rows_per_sub
        pltpu.sync_copy(z_hbm, acc_sh.at[pl.ds(row0, rows_per_sub)])
        pltpu.sync_copy(src_hbm.at[cid, sid], src_v)
        pltpu.sync_copy(dst_hbm.at[cid, sid], dst_v)
        plsc.subcore_barrier()

        @pl.loop(0, k_steps)
        def _(j):
            pltpu.sync_copy(g_hbm.at[src_v.at[j]], gbuf)
            pltpu.sync_copy(gbuf, acc_sh.at[dst_v.at[j]], add=True)

        plsc.subcore_barrier()
        pltpu.sync_copy(
            acc_sh.at[pl.ds(row0, rows_per_sub)],
            out_hbm.at[cid, pl.ds(row0, rows_per_sub)],
        )

    return agg_kernel(g, src_r, dst_r, zeros_d)


def _dinv_of(p0, p1):
    return lax.rsqrt(p0[:, 0] + p1[:, 0] + 1.0)


def _tc_g1(x, w1, p0, p1, block_rows=2000):
    n, d_in = x.shape
    d_hid = w1.shape[1]

    def body(x_ref, w_ref, p0_ref, p1_ref, g_ref):
        dinv = _dinv_of(p0_ref[...], p1_ref[...])
        h = jnp.dot(x_ref[...], w_ref[...],
                    preferred_element_type=jnp.float32,
                    precision=lax.Precision.HIGHEST)
        g_ref[...] = h * dinv[:, None]

    return pl.pallas_call(
        body,
        grid=(n // block_rows,),
        in_specs=[
            pl.BlockSpec((block_rows, d_in), lambda i: (i, 0)),
            pl.BlockSpec((d_in, d_hid), lambda i: (0, 0)),
            pl.BlockSpec((block_rows, 16), lambda i: (i, 0)),
            pl.BlockSpec((block_rows, 16), lambda i: (i, 0)),
        ],
        out_specs=pl.BlockSpec((block_rows, d_hid), lambda i: (i, 0)),
        out_shape=jax.ShapeDtypeStruct((n, d_hid), jnp.float32),
    )(x, w1, p0, p1)


def _tc_mid(a0, a1, g1, p0, p1, b1, w2, block_rows=2000):
    n, d_hid = g1.shape
    d_out = w2.shape[1]

    def body(a0_ref, a1_ref, g1_ref, p0_ref, p1_ref, b_ref, w_ref, g2_ref):
        dinv = _dinv_of(p0_ref[...], p1_ref[...])
        u = (a0_ref[...] + a1_ref[...] + g1_ref[...]) * dinv[:, None]
        u = jnp.maximum(u + b_ref[...], 0.0)
        h2 = jnp.dot(u, w_ref[...],
                     preferred_element_type=jnp.float32,
                     precision=lax.Precision.HIGHEST)
        g2_ref[...] = h2 * dinv[:, None]

    return pl.pallas_call(
        body,
        grid=(n // block_rows,),
        in_specs=[
            pl.BlockSpec((block_rows, d_hid), lambda i: (i, 0)),
            pl.BlockSpec((block_rows, d_hid), lambda i: (i, 0)),
            pl.BlockSpec((block_rows, d_hid), lambda i: (i, 0)),
            pl.BlockSpec((block_rows, 16), lambda i: (i, 0)),
            pl.BlockSpec((block_rows, 16), lambda i: (i, 0)),
            pl.BlockSpec((1, d_hid), lambda i: (0, 0)),
            pl.BlockSpec((d_hid, d_out), lambda i: (0, 0)),
        ],
        out_specs=pl.BlockSpec((block_rows, d_out), lambda i: (i, 0)),
        out_shape=jax.ShapeDtypeStruct((n, d_out), jnp.float32),
    )(a0, a1, g1, p0, p1, b1.reshape(1, d_hid), w2)


def _tc_out(a0, a1, g2, p0, p1, b2, block_rows=2000):
    n, d_out = g2.shape

    def body(a0_ref, a1_ref, g2_ref, p0_ref, p1_ref, b_ref, o_ref):
        dinv = _dinv_of(p0_ref[...], p1_ref[...])
        o_ref[...] = (a0_ref[...] + a1_ref[...] + g2_ref[...]) * dinv[:, None] \
            + b_ref[...]

    return pl.pallas_call(
        body,
        grid=(n // block_rows,),
        in_specs=[
            pl.BlockSpec((block_rows, d_out), lambda i: (i, 0)),
            pl.BlockSpec((block_rows, d_out), lambda i: (i, 0)),
            pl.BlockSpec((block_rows, d_out), lambda i: (i, 0)),
            pl.BlockSpec((block_rows, 16), lambda i: (i, 0)),
            pl.BlockSpec((block_rows, 16), lambda i: (i, 0)),
            pl.BlockSpec((1, d_out), lambda i: (0, 0)),
        ],
        out_specs=pl.BlockSpec((block_rows, d_out), lambda i: (i, 0)),
        out_shape=jax.ShapeDtypeStruct((n, d_out), jnp.float32),
    )(a0, a1, g2, p0, p1, b2.reshape(1, d_out))


def kernel(x, edge_index, W1, b1, W2, b2):
    n = x.shape[0]
    e = edge_index.shape[1]
    ei = edge_index.astype(jnp.int32)

    tiles = NC * NS
    k_steps = -(-e // (tiles * EPR))
    e_pad = tiles * k_steps * EPR
    src_p = jnp.concatenate(
        [ei[0], jnp.zeros((e_pad - e,), jnp.int32)]).reshape(NC, NS, k_steps, EPR)
    dst_p = jnp.concatenate(
        [ei[1], jnp.full((e_pad - e,), n, jnp.int32)]).reshape(NC, NS, k_steps, EPR)

    rows_per_sub = -(-(n + 1) // NS)
    rows_per_sub += rows_per_sub % 2
    r_pad = rows_per_sub * NS

    z16 = jnp.zeros((rows_per_sub, 16), jnp.float32)
    ones16 = jnp.ones((EPR, 16), jnp.float32)
    degp = _sc_degree(dst_p, z16, ones16, r_pad, rows_per_sub)
    p0, p1 = degp[0, :n, :], degp[1, :n, :]

    g1 = _tc_g1(x, W1, p0, p1)

    d_hid = W1.shape[1]
    acc1 = _sc_aggregate(g1, src_p, dst_p,
                         jnp.zeros((rows_per_sub, d_hid), jnp.float32),
                         r_pad, rows_per_sub)
    g2 = _tc_mid(acc1[0, :n], acc1[1, :n], g1, p0, p1, b1, W2)

    d_out = W2.shape[1]
    acc2 = _sc_aggregate(g2, src_p, dst_p,
                         jnp.zeros((rows_per_sub, d_out), jnp.float32),
                         r_pad, rows_per_sub)
    return _tc_out(acc2[0, :n], acc2[1, :n], g2, p0, p1, b2)


# speedup vs baseline: 12.1321x; 12.1321x over previous
import functools

import jax
import jax.numpy as jnp
from jax import lax
from jax.experimental import pallas as pl
from jax.experimental.pallas import tpu as pltpu
from jax.experimental.pallas import tpu_sc as plsc

NC = 2
NS = 16
EPR = 128


def _mesh():
    return plsc.VectorSubcoreMesh(
        core_axis_name="c", subcore_axis_name="s", num_cores=NC, num_subcores=NS
    )


def _sc_degree(dst_r, zeros128, ones128, r_pad, rows_per_sub):
    k_steps = dst_r.shape[2]

    @functools.partial(
        pl.kernel,
        out_type=jax.ShapeDtypeStruct((NC, r_pad, 128), jnp.float32),
        mesh=_mesh(),
        scratch_types=[
            pltpu.VMEM((k_steps, EPR), jnp.int32),
            pltpu.VMEM((EPR, 128), jnp.float32),
            pltpu.VMEM_SHARED((r_pad, 128), jnp.float32),
        ],
    )
    def deg_kernel(dst_hbm, z_hbm, one_hbm, out_hbm, dst_v, one_v, acc_sh):
        cid = lax.axis_index("c")
        sid = lax.axis_index("s")
        row0 = sid * rows_per_sub
        pltpu.sync_copy(z_hbm, acc_sh.at[pl.ds(row0, rows_per_sub)])
        pltpu.sync_copy(one_hbm, one_v)
        pltpu.sync_copy(dst_hbm.at[cid, sid], dst_v)
        plsc.subcore_barrier()

        @pl.loop(0, k_steps)
        def _(j):
            pltpu.sync_copy(one_v, acc_sh.at[dst_v.at[j]], add=True)

        plsc.subcore_barrier()
        pltpu.sync_copy(
            acc_sh.at[pl.ds(row0, rows_per_sub)],
            out_hbm.at[cid, pl.ds(row0, rows_per_sub)],
        )

    return deg_kernel(dst_r, zeros128, ones128)


def _sc_aggregate(g, src_r, dst_r, zeros_d, r_pad, rows_per_sub):
    d = g.shape[1]
    k_steps = src_r.shape[2]

    @functools.partial(
        pl.kernel,
        out_type=jax.ShapeDtypeStruct((NC, r_pad, d), jnp.float32),
        mesh=_mesh(),
        scratch_types=[
            pltpu.VMEM((k_steps, EPR), jnp.int32),
            pltpu.VMEM((k_steps, EPR), jnp.int32),
            pltpu.VMEM((EPR, d), jnp.float32),
            pltpu.VMEM_SHARED((r_pad, d), jnp.float32),
        ],
    )
    def agg_kernel(g_hbm, src_hbm, dst_hbm, z_hbm, out_hbm,
                   src_v, dst_v, gbuf, acc_sh):
        cid = lax.axis_index("c")
        sid = lax.axis_index("s")
        row0 = sid * rows_per_sub
        pltpu.sync_copy(z_hbm, acc_sh.at[pl.ds(row0, rows_per_sub)])
        pltpu.sync_copy(src_hbm.at[cid, sid], src_v)
        pltpu.sync_copy(dst_hbm.at[cid, sid], dst_v)
        plsc.subcore_barrier()

        @pl.loop(0, k_steps)
        def _(j):
            pltpu.sync_copy(g_hbm.at[src_v.at[j]], gbuf)
            pltpu.sync_copy(gbuf, acc_sh.at[dst_v.at[j]], add=True)

        plsc.subcore_barrier()
        pltpu.sync_copy(
            acc_sh.at[pl.ds(row0, rows_per_sub)],
            out_hbm.at[cid, pl.ds(row0, rows_per_sub)],
        )

    return agg_kernel(g, src_r, dst_r, zeros_d)


def _dinv_of(p0, p1):
    return lax.rsqrt(p0[:, 0] + p1[:, 0] + 1.0)


def _tc_g1(x, w1, p0, p1, block_rows=2000):
    n, d_in = x.shape
    d_hid = w1.shape[1]

    def body(x_ref, w_ref, p0_ref, p1_ref, g_ref):
        dinv = _dinv_of(p0_ref[...], p1_ref[...])
        h = jnp.dot(x_ref[...], w_ref[...],
                    preferred_element_type=jnp.float32,
                    precision=lax.Precision.HIGHEST)
        g_ref[...] = h * dinv[:, None]

    return pl.pallas_call(
        body,
        grid=(n // block_rows,),
        in_specs=[
            pl.BlockSpec((block_rows, d_in), lambda i: (i, 0)),
            pl.BlockSpec((d_in, d_hid), lambda i: (0, 0)),
            pl.BlockSpec((block_rows, 16), lambda i: (i, 0)),
            pl.BlockSpec((block_rows, 16), lambda i: (i, 0)),
        ],
        out_specs=pl.BlockSpec((block_rows, d_hid), lambda i: (i, 0)),
        out_shape=jax.ShapeDtypeStruct((n, d_hid), jnp.float32),
    )(x, w1, p0, p1)


def _tc_mid(a0, a1, g1, p0, p1, b1, w2, block_rows=2000):
    n, d_hid = g1.shape
    d_out = w2.shape[1]

    def body(a0_ref, a1_ref, g1_ref, p0_ref, p1_ref, b_ref, w_ref, g2_ref):
        dinv = _dinv_of(p0_ref[...], p1_ref[...])
        u = (a0_ref[...] + a1_ref[...] + g1_ref[...]) * dinv[:, None]
        u = jnp.maximum(u + b_ref[...], 0.0)
        h2 = jnp.dot(u, w_ref[...],
                     preferred_element_type=jnp.float32,
                     precision=lax.Precision.HIGHEST)
        g2_ref[...] = h2 * dinv[:, None]

    return pl.pallas_call(
        body,
        grid=(n // block_rows,),
        in_specs=[
            pl.BlockSpec((block_rows, d_hid), lambda i: (i, 0)),
            pl.BlockSpec((block_rows, d_hid), lambda i: (i, 0)),
            pl.BlockSpec((block_rows, d_hid), lambda i: (i, 0)),
            pl.BlockSpec((block_rows, 16), lambda i: (i, 0)),
            pl.BlockSpec((block_rows, 16), lambda i: (i, 0)),
            pl.BlockSpec((1, d_hid), lambda i: (0, 0)),
            pl.BlockSpec((d_hid, d_out), lambda i: (0, 0)),
        ],
        out_specs=pl.BlockSpec((block_rows, d_out), lambda i: (i, 0)),
        out_shape=jax.ShapeDtypeStruct((n, d_out), jnp.float32),
    )(a0, a1, g1, p0, p1, b1.reshape(1, d_hid), w2)


def _tc_out(a0, a1, g2, p0, p1, b2, block_rows=2000):
    n, d_out = g2.shape

    def body(a0_ref, a1_ref, g2_ref, p0_ref, p1_ref, b_ref, o_ref):
        dinv = _dinv_of(p0_ref[...], p1_ref[...])
        o_ref[...] = (a0_ref[...] + a1_ref[...] + g2_ref[...]) * dinv[:, None] \
            + b_ref[...]

    return pl.pallas_call(
        body,
        grid=(n // block_rows,),
        in_specs=[
            pl.BlockSpec((block_rows, d_out), lambda i: (i, 0)),
            pl.BlockSpec((block_rows, d_out), lambda i: (i, 0)),
            pl.BlockSpec((block_rows, d_out), lambda i: (i, 0)),
            pl.BlockSpec((block_rows, 16), lambda i: (i, 0)),
            pl.BlockSpec((block_rows, 16), lambda i: (i, 0)),
            pl.BlockSpec((1, d_out), lambda i: (0, 0)),
        ],
        out_specs=pl.BlockSpec((block_rows, d_out), lambda i: (i, 0)),
        out_shape=jax.ShapeDtypeStruct((n, d_out), jnp.float32),
    )(a0, a1, g2, p0, p1, b2.reshape(1, d_out))


def kernel(x, edge_index, W1, b1, W2, b2):
    n = x.shape[0]
    e = edge_index.shape[1]
    ei = edge_index.astype(jnp.int32)

    tiles = NC * NS
    k_steps = -(-e // (tiles * EPR))
    e_pad = tiles * k_steps * EPR
    src_p = jnp.concatenate(
        [ei[0], jnp.zeros((e_pad - e,), jnp.int32)]).reshape(NC, NS, k_steps, EPR)
    dst_p = jnp.concatenate(
        [ei[1], jnp.full((e_pad - e,), n, jnp.int32)]).reshape(NC, NS, k_steps, EPR)

    rows_per_sub = -(-(n + 1) // NS)
    rows_per_sub = -(-rows_per_sub // 8) * 8
    r_pad = rows_per_sub * NS

    z128 = jnp.zeros((rows_per_sub, 128), jnp.float32)
    ones128 = jnp.ones((EPR, 128), jnp.float32)
    degp = _sc_degree(dst_p, z128, ones128, r_pad, rows_per_sub)
    p0, p1 = degp[0, :n, :16], degp[1, :n, :16]

    g1 = _tc_g1(x, W1, p0, p1)

    d_hid = W1.shape[1]
    acc1 = _sc_aggregate(g1, src_p, dst_p,
                         jnp.zeros((rows_per_sub, d_hid), jnp.float32),
                         r_pad, rows_per_sub)
    d_out = W2.shape[1]
    w2p = jnp.pad(W2, ((0, 0), (0, 128 - d_out)))
    b2p = jnp.pad(b2, (0, 128 - d_out))
    g2 = _tc_mid(acc1[0, :n], acc1[1, :n], g1, p0, p1, b1, w2p)

    acc2 = _sc_aggregate(g2, src_p, dst_p,
                         jnp.zeros((rows_per_sub, 128), jnp.float32),
                         r_pad, rows_per_sub)
    out = _tc_out(acc2[0, :n], acc2[1, :n], g2, p0, p1, b2p)
    return out[:, :d_out]
